# baseline (speedup 1.0000x reference)
import math
import functools

import jax
import jax.numpy as jnp
import numpy as np
from jax.experimental import pallas as pl

B = 16; T = 2048; DIM = 16; NP = 8; RPP = 16; OBS = 1.0; PRED = 96
LAT = 128; HEADS = 8; DFF = 256; LAYERS = 3
HD = RPP // HEADS
NR = NP * RPP
ROWS = B * DIM
K1_BB = 4
K1_TC = 512


def _pe_lane():
    pos = np.arange(NP, dtype=np.float32)[:, None]
    div = np.exp(np.arange(0, RPP, 2, dtype=np.float32) * -(math.log(10000.0) / RPP))
    pe = np.zeros((NP, RPP), dtype=np.float32)
    pe[:, 0::2] = np.sin(pos * div)
    pe[:, 1::2] = np.cos(pos * div)
    return pe.reshape(1, NR)


def _selectors():
    Rm = np.zeros((NR, HEADS * NP), np.float32)
    Em = np.zeros((HEADS * NP, NR), np.float32)
    for j in range(NP):
        for h in range(HEADS):
            for u in range(HD):
                Rm[j * RPP + h * HD + u, h * NP + j] = 1.0
                Em[h * NP + j, j * RPP + h * HD + u] = 1.0
    Gm = np.kron(np.eye(HEADS, dtype=np.float32), np.ones((NP, NP), np.float32))
    Fm = np.tile(np.eye(RPP, dtype=np.float32), (NP, 1))
    Mm = np.kron(np.eye(NP, dtype=np.float32),
                 np.ones((RPP, RPP), np.float32) / RPP)
    Bm = np.kron(np.eye(NP, dtype=np.float32), np.ones((RPP, RPP), np.float32))
    Pm = np.zeros((NR, NR), np.float32)
    for p in range(NP):
        for r in range(RPP):
            Pm[p * RPP + r, r * NP + p] = 1.0
    return Rm, Em, Gm, Fm, Mm, Bm, Pm


_PE = _pe_lane()
_R, _E, _G, _F, _MAVG, _BMASK, _PERM = _selectors()
_X8 = np.kron(np.eye(NP, dtype=np.float32), np.ones((1, RPP), np.float32))
_T16 = np.tile(np.eye(RPP, dtype=np.float32), (1, NP))


def _k1_body(d_ref, w1_ref, b1_ref, w2_ref, b2_ref,
             lo_ref, hi_ref, x8_ref, rf_ref, out_ref):
    f32 = jnp.float32
    dgT = lambda a, b: jax.lax.dot_general(a, b, (((0,), (0,)), ((), ())),
                                           preferred_element_type=f32)
    for bb in range(K1_BB):
        datT = d_ref[bb]
        valsT = datT[:DIM, :]
        maskT = datT[DIM:2 * DIM, :]
        trow = datT[2 * DIM:2 * DIM + 1, :]
        tm8 = (trow >= lo_ref[...]) & (trow <= hi_ref[...])
        tm8t = jnp.transpose(tm8.astype(f32), (1, 0))
        tcol = jnp.transpose(trow, (1, 0))
        obs = (jnp.sum(maskT, axis=0, keepdims=True) > 0.0).astype(f32)
        parts = []
        for t0 in range(0, T, K1_TC):
            sl = slice(t0, t0 + K1_TC)
            vT = valsT[:, sl] * maskT[:, sl]
            h1 = jnp.maximum(dgT(w1_ref[...], vT) + b1_ref[...], 0.0)
            h = dgT(w2_ref[...], h1) + b2_ref[...]
            haug = jnp.concatenate([h, jnp.ones((1, K1_TC), f32)],
                                   axis=0) * obs[:, sl]
            tm = jnp.dot(tm8t[sl, :], x8_ref[...],
                         preferred_element_type=f32)
            u = rf_ref[...] - tcol[sl, :]
            w = jnp.exp(-100.0 * (u * u)) * tm
            parts.append(jnp.dot(haug, w, preferred_element_type=f32))
        repz = parts[0]
        for pz in parts[1:]:
            repz = repz + pz
        z = repz[DIM:DIM + 1, :]
        rz = 1.0 / jnp.where(z > 0.0, z, 1.0)
        out_ref[bb] = repz[:DIM, :] * rz


def _k2_body(x_ref, wq_ref, wk_ref, wv_ref, wo_ref, wf1_ref, bf1_ref,
             wf2_ref, bf2_ref, g1_ref, c1_ref, g2_ref, c2_ref, gf_ref, cf_ref,
             wl_ref, bl_ref, pe_ref, r_ref, e_ref, gg_ref, f_ref, mavg_ref,
             bm_ref, pm_ref, t16_ref, out_ref):
    f32 = jnp.float32
    dot = lambda a, b: jnp.dot(a, b, preferred_element_type=f32)
    lane8 = lambda a: jnp.concatenate([a] * NP, axis=1)
    def bdiag(w):
        wt = dot(w, t16_ref[...])
        return jnp.concatenate([wt] * NP, axis=0) * bm_ref[...]
    x = x_ref[...] + pe_ref[...]
    inv_sqrt_hd = 1.0 / math.sqrt(float(HD))

    g1t = [lane8(g1_ref[l:l + 1, :]) for l in range(LAYERS)]
    c1t = [lane8(c1_ref[l:l + 1, :]) for l in range(LAYERS)]
    g2t = [lane8(g2_ref[l:l + 1, :]) for l in range(LAYERS)]
    c2t = [lane8(c2_ref[l:l + 1, :]) for l in range(LAYERS)]
    fwo = [dot(f_ref[...], wo_ref[l]) for l in range(LAYERS)]

    def ln(z, g, c):
        m = dot(z, mavg_ref[...])
        ex2 = dot(z * z, mavg_ref[...])
        va = ex2 - m * m
        return (z - m) * jax.lax.rsqrt(va + 1e-5) * g + c

    for l in range(LAYERS):
        q = dot(x, bdiag(wq_ref[l]))
        k = dot(x, bdiag(wk_ref[l]))
        v = dot(x, bdiag(wv_ref[l]))
        o_parts = []
        for i in range(NP):
            qi = lane8(q[:, i * RPP:(i + 1) * RPP])
            si = dot(qi * k, r_ref[...])
            ei = jnp.exp(jnp.minimum(si, 60.0))
            di = dot(ei, gg_ref[...])
            pi = ei / di
            pex = dot(pi, e_ref[...])
            o_parts.append(dot(pex * v, fwo[l]))
        x = x + jnp.concatenate(o_parts, axis=1)
        x = ln(x, g1t[l], c1t[l])
        y_parts = []
        for p in range(NP):
            xp = x[:, p * RPP:(p + 1) * RPP]
            yp = jax.nn.gelu(dot(xp, wf1_ref[l]) + bf1_ref[l:l + 1, :])
            y_parts.append(dot(yp, wf2_ref[l]) + bf2_ref[l:l + 1, :])
        y = jnp.concatenate(y_parts, axis=1)
        x = ln(x + y, g2t[l], c2t[l])
    x = ln(x, lane8(gf_ref[...]), lane8(cf_ref[...]))
    wperm = dot(pm_ref[...], wl_ref[...])
    out_ref[...] = dot(x, wperm) + bl_ref[...]


@functools.partial(jax.jit, static_argnums=())
def kernel(data, W1, b1, W2, b2, Wq, Wk, Wv, Wo, Wf1, bf1, Wf2, bf2,
           ln1g, ln1b, ln2g, ln2b, lnfg, lnfb, Wlin, blin):
    f32 = jnp.float32
    edges = np.linspace(0.0, OBS, NP + 1)
    lo = jnp.asarray(edges[:NP], f32).reshape(NP, 1)
    hi = jnp.asarray(edges[1:], f32).reshape(NP, 1)
    refs = jnp.linspace(0.0, OBS, NR, dtype=f32).reshape(1, NR)
    dataT = jnp.transpose(data, (0, 2, 1))

    rep = pl.pallas_call(
        _k1_body,
        grid=(B // K1_BB,),
        in_specs=[
            pl.BlockSpec((K1_BB, 2 * DIM + 1, T), lambda b: (b, 0, 0)),
            pl.BlockSpec((DIM, LAT), lambda b: (0, 0)),
            pl.BlockSpec((LAT, 1), lambda b: (0, 0)),
            pl.BlockSpec((LAT, DIM), lambda b: (0, 0)),
            pl.BlockSpec((DIM, 1), lambda b: (0, 0)),
            pl.BlockSpec((NP, 1), lambda b: (0, 0)),
            pl.BlockSpec((NP, 1), lambda b: (0, 0)),
            pl.BlockSpec((NP, NR), lambda b: (0, 0)),
            pl.BlockSpec((1, NR), lambda b: (0, 0)),
        ],
        out_specs=pl.BlockSpec((K1_BB, DIM, NR), lambda b: (b, 0, 0)),
        out_shape=jax.ShapeDtypeStruct((B, DIM, NR), f32),
    )(dataT, W1, b1.reshape(LAT, 1), W2, b2.reshape(DIM, 1), lo, hi,
      jnp.asarray(_X8), refs)

    xflat = rep.reshape(ROWS, NR)

    full = lambda *sh: pl.BlockSpec(sh, lambda: tuple(0 for _ in sh))
    out = pl.pallas_call(
        _k2_body,
        in_specs=[
            full(ROWS, NR),
            full(LAYERS, RPP, RPP), full(LAYERS, RPP, RPP),
            full(LAYERS, RPP, RPP), full(LAYERS, RPP, RPP),
            full(LAYERS, RPP, DFF), full(LAYERS, DFF),
            full(LAYERS, DFF, RPP), full(LAYERS, RPP),
            full(LAYERS, RPP), full(LAYERS, RPP),
            full(LAYERS, RPP), full(LAYERS, RPP),
            full(1, RPP), full(1, RPP),
            full(NR, PRED), full(1, PRED),
            full(1, NR),
            full(NR, HEADS * NP), full(HEADS * NP, NR),
            full(HEADS * NP, HEADS * NP), full(NR, RPP),
            full(NR, NR), full(NR, NR), full(NR, NR),
            full(RPP, NR),
        ],
        out_specs=full(ROWS, PRED),
        out_shape=jax.ShapeDtypeStruct((ROWS, PRED), f32),
    )(xflat, Wq, Wk, Wv, Wo, Wf1, bf1, Wf2, bf2,
      ln1g, ln1b, ln2g, ln2b, lnfg.reshape(1, RPP), lnfb.reshape(1, RPP),
      Wlin, blin.reshape(1, PRED),
      jnp.asarray(_PE), jnp.asarray(_R / math.sqrt(float(HD))),
      jnp.asarray(_E), jnp.asarray(_G),
      jnp.asarray(_F), jnp.asarray(_MAVG), jnp.asarray(_BMASK),
      jnp.asarray(_PERM), jnp.asarray(_T16))

    return jnp.transpose(out.reshape(B, DIM, PRED), (0, 2, 1))


# speedup vs baseline: 1.1135x; 1.1135x over previous
import math
import functools

import jax
import jax.numpy as jnp
import numpy as np
from jax.experimental import pallas as pl
from jax.experimental.pallas import tpu as pltpu

B = 16; T = 2048; DIM = 16; NP = 8; RPP = 16; OBS = 1.0; PRED = 96
LAT = 128; HEADS = 8; DFF = 256; LAYERS = 3
HD = RPP // HEADS
NR = NP * RPP
ROWS = B * DIM
K1_BB = 4


def _pe_lane():
    pos = np.arange(NP, dtype=np.float32)[:, None]
    div = np.exp(np.arange(0, RPP, 2, dtype=np.float32) * -(math.log(10000.0) / RPP))
    pe = np.zeros((NP, RPP), dtype=np.float32)
    pe[:, 0::2] = np.sin(pos * div)
    pe[:, 1::2] = np.cos(pos * div)
    return pe.reshape(1, NR)


def _selectors():
    Rm = np.zeros((NR, HEADS * NP), np.float32)
    Em = np.zeros((HEADS * NP, NR), np.float32)
    for j in range(NP):
        for h in range(HEADS):
            for u in range(HD):
                Rm[j * RPP + h * HD + u, h * NP + j] = 1.0
                Em[h * NP + j, j * RPP + h * HD + u] = 1.0
    Gm = np.kron(np.eye(HEADS, dtype=np.float32), np.ones((NP, NP), np.float32))
    Fm = np.tile(np.eye(RPP, dtype=np.float32), (NP, 1))
    Mm = np.kron(np.eye(NP, dtype=np.float32),
                 np.ones((RPP, RPP), np.float32) / RPP)
    Bm = np.kron(np.eye(NP, dtype=np.float32), np.ones((RPP, RPP), np.float32))
    Pm = np.zeros((NR, NR), np.float32)
    for p in range(NP):
        for r in range(RPP):
            Pm[p * RPP + r, r * NP + p] = 1.0
    return Rm, Em, Gm, Fm, Mm, Bm, Pm


_PE = _pe_lane()
_R, _E, _G, _F, _MAVG, _BMASK, _PERM = _selectors()
_X8 = np.kron(np.eye(NP, dtype=np.float32), np.ones((1, RPP), np.float32))
_T16 = np.tile(np.eye(RPP, dtype=np.float32), (1, NP))


def _k1_phase(d_ref, w1_ref, b1_ref, w2_ref, b2_ref,
              lo_ref, hi_ref, x8_ref, rf_ref, acc_ref, base):
    f32 = jnp.float32
    dgT = lambda a, b: jax.lax.dot_general(a, b, (((0,), (0,)), ((), ())),
                                           preferred_element_type=f32)
    for bb in range(K1_BB):
        datT = d_ref[bb]
        valsT = datT[:DIM, :]
        maskT = datT[DIM:2 * DIM, :]
        trow = datT[2 * DIM:2 * DIM + 1, :]
        vT = valsT * maskT
        h1T = jnp.maximum(dgT(w1_ref[...], vT) + b1_ref[...], 0.0)
        hT = dgT(w2_ref[...], h1T) + b2_ref[...]
        obs = jnp.sum(maskT, axis=0, keepdims=True) > 0.0
        tm8 = ((trow >= lo_ref[...]) & (trow <= hi_ref[...]) & obs)
        tm8t = jnp.transpose(tm8.astype(f32), (1, 0))
        tm = jnp.dot(tm8t, x8_ref[...], preferred_element_type=f32)
        tcol = jnp.transpose(trow, (1, 0))
        u = rf_ref[...] - tcol
        w = jnp.exp(-100.0 * (u * u)) * tm
        haug = jnp.concatenate([hT, jnp.ones((1, T), f32)], axis=0)
        repz = jnp.dot(haug, w, preferred_element_type=f32)
        z = repz[DIM:DIM + 1, :]
        rz = 1.0 / jnp.where(z > 0.0, z, 1.0)
        acc_ref[pl.ds(base + bb * DIM, DIM), :] = repz[:DIM, :] * rz


def _k2_phase(x_ref, wq_ref, wk_ref, wv_ref, wo_ref, wf1_ref, bf1_ref,
             wf2_ref, bf2_ref, g1_ref, c1_ref, g2_ref, c2_ref, gf_ref, cf_ref,
             wl_ref, bl_ref, pe_ref, r_ref, e_ref, gg_ref, f_ref, mavg_ref,
             bm_ref, pm_ref, t16_ref, out_ref):
    f32 = jnp.float32
    dot = lambda a, b: jnp.dot(a, b, preferred_element_type=f32)
    lane8 = lambda a: jnp.concatenate([a] * NP, axis=1)
    def bdiag(w):
        wt = dot(w, t16_ref[...])
        return jnp.concatenate([wt] * NP, axis=0) * bm_ref[...]
    x = x_ref[...] + pe_ref[...]
    inv_sqrt_hd = 1.0 / math.sqrt(float(HD))

    g1t = [lane8(g1_ref[l:l + 1, :]) for l in range(LAYERS)]
    c1t = [lane8(c1_ref[l:l + 1, :]) for l in range(LAYERS)]
    g2t = [lane8(g2_ref[l:l + 1, :]) for l in range(LAYERS)]
    c2t = [lane8(c2_ref[l:l + 1, :]) for l in range(LAYERS)]
    fwo = [dot(f_ref[...], wo_ref[l]) for l in range(LAYERS)]

    def ln(z, g, c):
        m = dot(z, mavg_ref[...])
        ex2 = dot(z * z, mavg_ref[...])
        va = ex2 - m * m
        return (z - m) * jax.lax.rsqrt(va + 1e-5) * g + c

    for l in range(LAYERS):
        q = dot(x, bdiag(wq_ref[l]))
        k = dot(x, bdiag(wk_ref[l]))
        v = dot(x, bdiag(wv_ref[l]))
        o_parts = []
        for i in range(NP):
            qi = lane8(q[:, i * RPP:(i + 1) * RPP])
            si = dot(qi * k, r_ref[...])
            ei = jnp.exp(jnp.minimum(si, 60.0))
            di = dot(ei, gg_ref[...])
            pi = ei / di
            pex = dot(pi, e_ref[...])
            o_parts.append(dot(pex * v, fwo[l]))
        x = x + jnp.concatenate(o_parts, axis=1)
        x = ln(x, g1t[l], c1t[l])
        y_parts = []
        for p in range(NP):
            xp = x[:, p * RPP:(p + 1) * RPP]
            yp = jax.nn.gelu(dot(xp, wf1_ref[l]) + bf1_ref[l:l + 1, :])
            y_parts.append(dot(yp, wf2_ref[l]) + bf2_ref[l:l + 1, :])
        y = jnp.concatenate(y_parts, axis=1)
        x = ln(x + y, g2t[l], c2t[l])
    x = ln(x, lane8(gf_ref[...]), lane8(cf_ref[...]))
    wperm = dot(pm_ref[...], wl_ref[...])
    out_ref[...] = dot(x, wperm) + bl_ref[...]


def _fused_body(d_ref, w1_ref, b1_ref, w2_ref, b2_ref, lo_ref, hi_ref,
                x8_ref, rf_ref, wq_ref, wk_ref, wv_ref, wo_ref, wf1_ref,
                bf1_ref, wf2_ref, bf2_ref, g1_ref, c1_ref, g2_ref, c2_ref,
                gf_ref, cf_ref, wl_ref, bl_ref, pe_ref, r_ref, e_ref, gg_ref,
                f_ref, mavg_ref, bm_ref, pm_ref, t16_ref, out_ref, acc_ref):
    pid = pl.program_id(0)
    nk1 = B // K1_BB

    @pl.when(pid < nk1)
    def _():
        _k1_phase(d_ref, w1_ref, b1_ref, w2_ref, b2_ref, lo_ref, hi_ref,
                  x8_ref, rf_ref, acc_ref, pid * (K1_BB * DIM))

    @pl.when(pid == nk1)
    def _():
        _k2_phase(acc_ref, wq_ref, wk_ref, wv_ref, wo_ref, wf1_ref, bf1_ref,
                  wf2_ref, bf2_ref, g1_ref, c1_ref, g2_ref, c2_ref, gf_ref,
                  cf_ref, wl_ref, bl_ref, pe_ref, r_ref, e_ref, gg_ref,
                  f_ref, mavg_ref, bm_ref, pm_ref, t16_ref, out_ref)


@functools.partial(jax.jit, static_argnums=())
def kernel(data, W1, b1, W2, b2, Wq, Wk, Wv, Wo, Wf1, bf1, Wf2, bf2,
           ln1g, ln1b, ln2g, ln2b, lnfg, lnfb, Wlin, blin):
    f32 = jnp.float32
    edges = np.linspace(0.0, OBS, NP + 1)
    lo = jnp.asarray(edges[:NP], f32).reshape(NP, 1)
    hi = jnp.asarray(edges[1:], f32).reshape(NP, 1)
    refs = jnp.linspace(0.0, OBS, NR, dtype=f32).reshape(1, NR)
    dataT = jnp.transpose(data, (0, 2, 1))

    nk1 = B // K1_BB
    cm = lambda *sh: pl.BlockSpec(sh, lambda i: tuple(0 for _ in sh))
    out = pl.pallas_call(
        _fused_body,
        grid=(nk1 + 1,),
        in_specs=[
            pl.BlockSpec((K1_BB, 2 * DIM + 1, T),
                         lambda i: (jnp.minimum(i, nk1 - 1), 0, 0)),
            cm(DIM, LAT), cm(LAT, 1), cm(LAT, DIM), cm(DIM, 1),
            cm(NP, 1), cm(NP, 1), cm(NP, NR), cm(1, NR),
            cm(LAYERS, RPP, RPP), cm(LAYERS, RPP, RPP),
            cm(LAYERS, RPP, RPP), cm(LAYERS, RPP, RPP),
            cm(LAYERS, RPP, DFF), cm(LAYERS, DFF),
            cm(LAYERS, DFF, RPP), cm(LAYERS, RPP),
            cm(LAYERS, RPP), cm(LAYERS, RPP),
            cm(LAYERS, RPP), cm(LAYERS, RPP),
            cm(1, RPP), cm(1, RPP),
            cm(NR, PRED), cm(1, PRED),
            cm(1, NR),
            cm(NR, HEADS * NP), cm(HEADS * NP, NR),
            cm(HEADS * NP, HEADS * NP), cm(NR, RPP),
            cm(NR, NR), cm(NR, NR), cm(NR, NR),
            cm(RPP, NR),
        ],
        out_specs=cm(ROWS, PRED),
        out_shape=jax.ShapeDtypeStruct((ROWS, PRED), f32),
        scratch_shapes=[pltpu.VMEM((ROWS, NR), f32)],
    )(dataT, W1, b1.reshape(LAT, 1), W2, b2.reshape(DIM, 1), lo, hi,
      jnp.asarray(_X8), refs,
      Wq, Wk, Wv, Wo, Wf1, bf1, Wf2, bf2,
      ln1g, ln1b, ln2g, ln2b, lnfg.reshape(1, RPP), lnfb.reshape(1, RPP),
      Wlin, blin.reshape(1, PRED),
      jnp.asarray(_PE), jnp.asarray(_R / math.sqrt(float(HD))),
      jnp.asarray(_E), jnp.asarray(_G),
      jnp.asarray(_F), jnp.asarray(_MAVG), jnp.asarray(_BMASK),
      jnp.asarray(_PERM), jnp.asarray(_T16))

    return jnp.transpose(out.reshape(B, DIM, PRED), (0, 2, 1))


# speedup vs baseline: 1.1639x; 1.0453x over previous
import math
import functools

import jax
import jax.numpy as jnp
import numpy as np
from jax.experimental import pallas as pl

B = 16; T = 2048; DIM = 16; NP = 8; RPP = 16; OBS = 1.0; PRED = 96
LAT = 128; HEADS = 8; DFF = 256; LAYERS = 3
HD = RPP // HEADS
NR = NP * RPP
ROWS = B * DIM
K1_BB = 8


def _pe_lane():
    pos = np.arange(NP, dtype=np.float32)[:, None]
    div = np.exp(np.arange(0, RPP, 2, dtype=np.float32) * -(math.log(10000.0) / RPP))
    pe = np.zeros((NP, RPP), dtype=np.float32)
    pe[:, 0::2] = np.sin(pos * div)
    pe[:, 1::2] = np.cos(pos * div)
    return pe.reshape(1, NR)


def _selectors():
    Rm = np.zeros((NR, HEADS * NP), np.float32)
    Em = np.zeros((HEADS * NP, NR), np.float32)
    for j in range(NP):
        for h in range(HEADS):
            for u in range(HD):
                Rm[j * RPP + h * HD + u, h * NP + j] = 1.0
                Em[h * NP + j, j * RPP + h * HD + u] = 1.0
    Gm = np.kron(np.eye(HEADS, dtype=np.float32), np.ones((NP, NP), np.float32))
    Fm = np.tile(np.eye(RPP, dtype=np.float32), (NP, 1))
    Mm = np.kron(np.eye(NP, dtype=np.float32),
                 np.ones((RPP, RPP), np.float32) / RPP)
    Bm = np.kron(np.eye(NP, dtype=np.float32), np.ones((RPP, RPP), np.float32))
    Pm = np.zeros((NR, NR), np.float32)
    for p in range(NP):
        for r in range(RPP):
            Pm[p * RPP + r, r * NP + p] = 1.0
    return Rm, Em, Gm, Fm, Mm, Bm, Pm


_PE = _pe_lane()
_R, _E, _G, _F, _MAVG, _BMASK, _PERM = _selectors()
_X8 = np.kron(np.eye(NP, dtype=np.float32), np.ones((1, RPP), np.float32))
_T16 = np.tile(np.eye(RPP, dtype=np.float32), (1, NP))


def _k1_body(d_ref, w1_ref, b1_ref, w2_ref, b2_ref,
             lo_ref, hi_ref, x8_ref, rf_ref, out_ref):
    f32 = jnp.float32
    dgT = lambda a, b: jax.lax.dot_general(a, b, (((0,), (0,)), ((), ())),
                                           preferred_element_type=f32)
    for bb in range(K1_BB):
        datT = d_ref[bb]
        valsT = datT[:DIM, :]
        maskT = datT[DIM:2 * DIM, :]
        trow = datT[2 * DIM:2 * DIM + 1, :]
        vT = valsT * maskT
        h1T = jnp.maximum(dgT(w1_ref[...], vT) + b1_ref[...], 0.0)
        hT = dgT(w2_ref[...], h1T) + b2_ref[...]
        obs = jnp.sum(maskT, axis=0, keepdims=True) > 0.0
        tm8 = ((trow >= lo_ref[...]) & (trow <= hi_ref[...]) & obs)
        tm8t = jnp.transpose(tm8.astype(f32), (1, 0))
        tm = jnp.dot(tm8t, x8_ref[...], preferred_element_type=f32)
        tcol = jnp.transpose(trow, (1, 0))
        u = rf_ref[...] - tcol
        w = jnp.exp(-100.0 * (u * u)) * tm
        haug = jnp.concatenate([hT, jnp.ones((1, T), f32)], axis=0)
        repz = jnp.dot(haug, w, preferred_element_type=f32)
        z = repz[DIM:DIM + 1, :]
        rz = 1.0 / jnp.where(z > 0.0, z, 1.0)
        out_ref[bb] = repz[:DIM, :] * rz


def _k2_body(x_ref, wq_ref, wk_ref, wv_ref, wo_ref, wf1_ref, bf1_ref,
             wf2_ref, bf2_ref, g1_ref, c1_ref, g2_ref, c2_ref, gf_ref, cf_ref,
             wl_ref, bl_ref, pe_ref, r_ref, e_ref, gg_ref, f_ref, mavg_ref,
             bm_ref, pm_ref, t16_ref, out_ref):
    f32 = jnp.float32
    dot = lambda a, b: jnp.dot(a, b, preferred_element_type=f32)
    lane8 = lambda a: jnp.concatenate([a] * NP, axis=1)
    def bdiag(w):
        wt = dot(w, t16_ref[...])
        return jnp.concatenate([wt] * NP, axis=0) * bm_ref[...]
    x = x_ref[...] + pe_ref[...]
    inv_sqrt_hd = 1.0 / math.sqrt(float(HD))

    g1t = [lane8(g1_ref[l:l + 1, :]) for l in range(LAYERS)]
    c1t = [lane8(c1_ref[l:l + 1, :]) for l in range(LAYERS)]
    g2t = [lane8(g2_ref[l:l + 1, :]) for l in range(LAYERS)]
    c2t = [lane8(c2_ref[l:l + 1, :]) for l in range(LAYERS)]
    fwo = [dot(f_ref[...], wo_ref[l]) for l in range(LAYERS)]

    def ln(z, g, c):
        m = dot(z, mavg_ref[...])
        ex2 = dot(z * z, mavg_ref[...])
        va = ex2 - m * m
        return (z - m) * jax.lax.rsqrt(va + 1e-5) * g + c

    for l in range(LAYERS):
        q = dot(x, bdiag(wq_ref[l]))
        k = dot(x, bdiag(wk_ref[l]))
        v = dot(x, bdiag(wv_ref[l]))
        o_parts = []
        for i in range(NP):
            qi = lane8(q[:, i * RPP:(i + 1) * RPP])
            si = dot(qi * k, r_ref[...])
            ei = jnp.exp(jnp.minimum(si, 60.0))
            di = dot(ei, gg_ref[...])
            pi = ei / di
            pex = dot(pi, e_ref[...])
            o_parts.append(dot(pex * v, fwo[l]))
        x = x + jnp.concatenate(o_parts, axis=1)
        x = ln(x, g1t[l], c1t[l])
        y_parts = []
        for p in range(NP):
            xp = x[:, p * RPP:(p + 1) * RPP]
            yp = jax.nn.gelu(dot(xp, wf1_ref[l]) + bf1_ref[l:l + 1, :])
            y_parts.append(dot(yp, wf2_ref[l]) + bf2_ref[l:l + 1, :])
        y = jnp.concatenate(y_parts, axis=1)
        x = ln(x + y, g2t[l], c2t[l])
    x = ln(x, lane8(gf_ref[...]), lane8(cf_ref[...]))
    wperm = dot(pm_ref[...], wl_ref[...])
    out_ref[...] = dot(x, wperm) + bl_ref[...]


@functools.partial(jax.jit, static_argnums=())
def kernel(data, W1, b1, W2, b2, Wq, Wk, Wv, Wo, Wf1, bf1, Wf2, bf2,
           ln1g, ln1b, ln2g, ln2b, lnfg, lnfb, Wlin, blin):
    f32 = jnp.float32
    edges = np.linspace(0.0, OBS, NP + 1)
    lo = jnp.asarray(edges[:NP], f32).reshape(NP, 1)
    hi = jnp.asarray(edges[1:], f32).reshape(NP, 1)
    refs = jnp.linspace(0.0, OBS, NR, dtype=f32).reshape(1, NR)
    dataT = jnp.transpose(data, (0, 2, 1))

    rep = pl.pallas_call(
        _k1_body,
        grid=(B // K1_BB,),
        in_specs=[
            pl.BlockSpec((K1_BB, 2 * DIM + 1, T), lambda b: (b, 0, 0)),
            pl.BlockSpec((DIM, LAT), lambda b: (0, 0)),
            pl.BlockSpec((LAT, 1), lambda b: (0, 0)),
            pl.BlockSpec((LAT, DIM), lambda b: (0, 0)),
            pl.BlockSpec((DIM, 1), lambda b: (0, 0)),
            pl.BlockSpec((NP, 1), lambda b: (0, 0)),
            pl.BlockSpec((NP, 1), lambda b: (0, 0)),
            pl.BlockSpec((NP, NR), lambda b: (0, 0)),
            pl.BlockSpec((1, NR), lambda b: (0, 0)),
        ],
        out_specs=pl.BlockSpec((K1_BB, DIM, NR), lambda b: (b, 0, 0)),
        out_shape=jax.ShapeDtypeStruct((B, DIM, NR), f32),
    )(dataT, W1, b1.reshape(LAT, 1), W2, b2.reshape(DIM, 1), lo, hi,
      jnp.asarray(_X8), refs)

    xflat = rep.reshape(ROWS, NR)

    full = lambda *sh: pl.BlockSpec(sh, lambda: tuple(0 for _ in sh))
    out = pl.pallas_call(
        _k2_body,
        in_specs=[
            full(ROWS, NR),
            full(LAYERS, RPP, RPP), full(LAYERS, RPP, RPP),
            full(LAYERS, RPP, RPP), full(LAYERS, RPP, RPP),
            full(LAYERS, RPP, DFF), full(LAYERS, DFF),
            full(LAYERS, DFF, RPP), full(LAYERS, RPP),
            full(LAYERS, RPP), full(LAYERS, RPP),
            full(LAYERS, RPP), full(LAYERS, RPP),
            full(1, RPP), full(1, RPP),
            full(NR, PRED), full(1, PRED),
            full(1, NR),
            full(NR, HEADS * NP), full(HEADS * NP, NR),
            full(HEADS * NP, HEADS * NP), full(NR, RPP),
            full(NR, NR), full(NR, NR), full(NR, NR),
            full(RPP, NR),
        ],
        out_specs=full(ROWS, PRED),
        out_shape=jax.ShapeDtypeStruct((ROWS, PRED), f32),
    )(xflat, Wq, Wk, Wv, Wo, Wf1, bf1, Wf2, bf2,
      ln1g, ln1b, ln2g, ln2b, lnfg.reshape(1, RPP), lnfb.reshape(1, RPP),
      Wlin, blin.reshape(1, PRED),
      jnp.asarray(_PE), jnp.asarray(_R / math.sqrt(float(HD))),
      jnp.asarray(_E), jnp.asarray(_G),
      jnp.asarray(_F), jnp.asarray(_MAVG), jnp.asarray(_BMASK),
      jnp.asarray(_PERM), jnp.asarray(_T16))

    return jnp.transpose(out.reshape(B, DIM, PRED), (0, 2, 1))


# speedup vs baseline: 1.1803x; 1.0141x over previous
import math
import functools

import jax
import jax.numpy as jnp
import numpy as np
from jax.experimental import pallas as pl
from jax.experimental.pallas import tpu as pltpu

B = 16; T = 2048; DIM = 16; NP = 8; RPP = 16; OBS = 1.0; PRED = 96
LAT = 128; HEADS = 8; DFF = 256; LAYERS = 3
HD = RPP // HEADS
NR = NP * RPP
ROWS = B * DIM
K1_BB = 4


def _pe_lane():
    pos = np.arange(NP, dtype=np.float32)[:, None]
    div = np.exp(np.arange(0, RPP, 2, dtype=np.float32) * -(math.log(10000.0) / RPP))
    pe = np.zeros((NP, RPP), dtype=np.float32)
    pe[:, 0::2] = np.sin(pos * div)
    pe[:, 1::2] = np.cos(pos * div)
    return pe.reshape(1, NR)


def _selectors():
    Rm = np.zeros((NR, HEADS * NP), np.float32)
    Em = np.zeros((HEADS * NP, NR), np.float32)
    for j in range(NP):
        for h in range(HEADS):
            for u in range(HD):
                Rm[j * RPP + h * HD + u, h * NP + j] = 1.0
                Em[h * NP + j, j * RPP + h * HD + u] = 1.0
    Gm = np.kron(np.eye(HEADS, dtype=np.float32), np.ones((NP, NP), np.float32))
    Fm = np.tile(np.eye(RPP, dtype=np.float32), (NP, 1))
    Mm = np.kron(np.eye(NP, dtype=np.float32),
                 np.ones((RPP, RPP), np.float32) / RPP)
    Bm = np.kron(np.eye(NP, dtype=np.float32), np.ones((RPP, RPP), np.float32))
    Pm = np.zeros((NR, NR), np.float32)
    for p in range(NP):
        for r in range(RPP):
            Pm[p * RPP + r, r * NP + p] = 1.0
    return Rm, Em, Gm, Fm, Mm, Bm, Pm


_PE = _pe_lane()
_R, _E, _G, _F, _MAVG, _BMASK, _PERM = _selectors()
_X8 = np.kron(np.eye(NP, dtype=np.float32), np.ones((1, RPP), np.float32))
_T16 = np.tile(np.eye(RPP, dtype=np.float32), (1, NP))


def _k1_body(d_ref, w1_ref, b1_ref, w2_ref, b2_ref,
             lo_ref, hi_ref, x8_ref, rf_ref, out_ref):
    f32 = jnp.float32
    dgT = lambda a, b: jax.lax.dot_general(a, b, (((0,), (0,)), ((), ())),
                                           preferred_element_type=f32)
    for bb in range(K1_BB):
        datT = d_ref[bb]
        valsT = datT[:DIM, :]
        maskT = datT[DIM:2 * DIM, :]
        trow = datT[2 * DIM:2 * DIM + 1, :]
        vT = valsT * maskT
        h1T = jnp.maximum(dgT(w1_ref[...], vT) + b1_ref[...], 0.0)
        hT = dgT(w2_ref[...], h1T) + b2_ref[...]
        obs = jnp.sum(maskT, axis=0, keepdims=True) > 0.0
        tm8 = ((trow >= lo_ref[...]) & (trow <= hi_ref[...]) & obs)
        tm8t = jnp.transpose(tm8.astype(f32), (1, 0))
        tm = jnp.dot(tm8t, x8_ref[...], preferred_element_type=f32)
        tcol = jnp.transpose(trow, (1, 0))
        u = rf_ref[...] - tcol
        w = jnp.exp(-100.0 * (u * u)) * tm
        haug = jnp.concatenate([hT, jnp.ones((1, T), f32)], axis=0)
        repz = jnp.dot(haug, w, preferred_element_type=f32)
        z = repz[DIM:DIM + 1, :]
        rz = 1.0 / jnp.where(z > 0.0, z, 1.0)
        out_ref[bb] = repz[:DIM, :] * rz


def _k2_body(x_ref, wq_ref, wk_ref, wv_ref, wo_ref, wf1_ref, bf1_ref,
             wf2_ref, bf2_ref, g1_ref, c1_ref, g2_ref, c2_ref, gf_ref, cf_ref,
             wl_ref, bl_ref, pe_ref, r_ref, e_ref, gg_ref, f_ref, mavg_ref,
             bm_ref, pm_ref, t16_ref, out_ref):
    f32 = jnp.float32
    dot = lambda a, b: jnp.dot(a, b, preferred_element_type=f32)
    lane8 = lambda a: jnp.concatenate([a] * NP, axis=1)
    def bdiag(w):
        wt = dot(w, t16_ref[...])
        return jnp.concatenate([wt] * NP, axis=0) * bm_ref[...]
    x = x_ref[...] + pe_ref[...]
    inv_sqrt_hd = 1.0 / math.sqrt(float(HD))

    g1t = [lane8(g1_ref[l:l + 1, :]) for l in range(LAYERS)]
    c1t = [lane8(c1_ref[l:l + 1, :]) for l in range(LAYERS)]
    g2t = [lane8(g2_ref[l:l + 1, :]) for l in range(LAYERS)]
    c2t = [lane8(c2_ref[l:l + 1, :]) for l in range(LAYERS)]
    fwo = [dot(f_ref[...], wo_ref[l]) for l in range(LAYERS)]

    def ln(z, g, c):
        m = dot(z, mavg_ref[...])
        ex2 = dot(z * z, mavg_ref[...])
        va = ex2 - m * m
        return (z - m) * jax.lax.rsqrt(va + 1e-5) * g + c

    for l in range(LAYERS):
        q = dot(x, bdiag(wq_ref[l]))
        k = dot(x, bdiag(wk_ref[l]))
        v = dot(x, bdiag(wv_ref[l]))
        o_parts = []
        for i in range(NP):
            qi = lane8(q[:, i * RPP:(i + 1) * RPP])
            si = dot(qi * k, r_ref[...])
            ei = jnp.exp(jnp.minimum(si, 60.0))
            di = dot(ei, gg_ref[...])
            pi = ei / di
            pex = dot(pi, e_ref[...])
            o_parts.append(dot(pex * v, fwo[l]))
        x = x + jnp.concatenate(o_parts, axis=1)
        x = ln(x, g1t[l], c1t[l])
        y_parts = []
        for p in range(NP):
            xp = x[:, p * RPP:(p + 1) * RPP]
            yp = jax.nn.gelu(dot(xp, wf1_ref[l]) + bf1_ref[l:l + 1, :])
            y_parts.append(dot(yp, wf2_ref[l]) + bf2_ref[l:l + 1, :])
        y = jnp.concatenate(y_parts, axis=1)
        x = ln(x + y, g2t[l], c2t[l])
    x = ln(x, lane8(gf_ref[...]), lane8(cf_ref[...]))
    wperm = dot(pm_ref[...], wl_ref[...])
    out_ref[...] = dot(x, wperm) + bl_ref[...]


@functools.partial(jax.jit, static_argnums=())
def kernel(data, W1, b1, W2, b2, Wq, Wk, Wv, Wo, Wf1, bf1, Wf2, bf2,
           ln1g, ln1b, ln2g, ln2b, lnfg, lnfb, Wlin, blin):
    f32 = jnp.float32
    edges = np.linspace(0.0, OBS, NP + 1)
    lo = jnp.asarray(edges[:NP], f32).reshape(NP, 1)
    hi = jnp.asarray(edges[1:], f32).reshape(NP, 1)
    refs = jnp.linspace(0.0, OBS, NR, dtype=f32).reshape(1, NR)
    dataT = jnp.transpose(data, (0, 2, 1))

    rep = pl.pallas_call(
        _k1_body,
        grid=(B // K1_BB,),
        in_specs=[
            pl.BlockSpec((K1_BB, 2 * DIM + 1, T), lambda b: (b, 0, 0)),
            pl.BlockSpec((DIM, LAT), lambda b: (0, 0)),
            pl.BlockSpec((LAT, 1), lambda b: (0, 0)),
            pl.BlockSpec((LAT, DIM), lambda b: (0, 0)),
            pl.BlockSpec((DIM, 1), lambda b: (0, 0)),
            pl.BlockSpec((NP, 1), lambda b: (0, 0)),
            pl.BlockSpec((NP, 1), lambda b: (0, 0)),
            pl.BlockSpec((NP, NR), lambda b: (0, 0)),
            pl.BlockSpec((1, NR), lambda b: (0, 0)),
        ],
        out_specs=pl.BlockSpec((K1_BB, DIM, NR), lambda b: (b, 0, 0)),
        out_shape=jax.ShapeDtypeStruct((B, DIM, NR), f32),
        compiler_params=pltpu.CompilerParams(
            dimension_semantics=("parallel",)),
    )(dataT, W1, b1.reshape(LAT, 1), W2, b2.reshape(DIM, 1), lo, hi,
      jnp.asarray(_X8), refs)

    xflat = rep.reshape(ROWS, NR)

    full = lambda *sh: pl.BlockSpec(sh, lambda: tuple(0 for _ in sh))
    out = pl.pallas_call(
        _k2_body,
        in_specs=[
            full(ROWS, NR),
            full(LAYERS, RPP, RPP), full(LAYERS, RPP, RPP),
            full(LAYERS, RPP, RPP), full(LAYERS, RPP, RPP),
            full(LAYERS, RPP, DFF), full(LAYERS, DFF),
            full(LAYERS, DFF, RPP), full(LAYERS, RPP),
            full(LAYERS, RPP), full(LAYERS, RPP),
            full(LAYERS, RPP), full(LAYERS, RPP),
            full(1, RPP), full(1, RPP),
            full(NR, PRED), full(1, PRED),
            full(1, NR),
            full(NR, HEADS * NP), full(HEADS * NP, NR),
            full(HEADS * NP, HEADS * NP), full(NR, RPP),
            full(NR, NR), full(NR, NR), full(NR, NR),
            full(RPP, NR),
        ],
        out_specs=full(ROWS, PRED),
        out_shape=jax.ShapeDtypeStruct((ROWS, PRED), f32),
    )(xflat, Wq, Wk, Wv, Wo, Wf1, bf1, Wf2, bf2,
      ln1g, ln1b, ln2g, ln2b, lnfg.reshape(1, RPP), lnfb.reshape(1, RPP),
      Wlin, blin.reshape(1, PRED),
      jnp.asarray(_PE), jnp.asarray(_R / math.sqrt(float(HD))),
      jnp.asarray(_E), jnp.asarray(_G),
      jnp.asarray(_F), jnp.asarray(_MAVG), jnp.asarray(_BMASK),
      jnp.asarray(_PERM), jnp.asarray(_T16))

    return jnp.transpose(out.reshape(B, DIM, PRED), (0, 2, 1))


# speedup vs baseline: 1.1832x; 1.0024x over previous
import math
import functools

import jax
import jax.numpy as jnp
import numpy as np
from jax.experimental import pallas as pl

B = 16; T = 2048; DIM = 16; NP = 8; RPP = 16; OBS = 1.0; PRED = 96
LAT = 128; HEADS = 8; DFF = 256; LAYERS = 3
HD = RPP // HEADS
NR = NP * RPP
ROWS = B * DIM
K1_BB = 4


def _pe_lane():
    pos = np.arange(NP, dtype=np.float32)[:, None]
    div = np.exp(np.arange(0, RPP, 2, dtype=np.float32) * -(math.log(10000.0) / RPP))
    pe = np.zeros((NP, RPP), dtype=np.float32)
    pe[:, 0::2] = np.sin(pos * div)
    pe[:, 1::2] = np.cos(pos * div)
    return pe.reshape(1, NR)


def _selectors():
    Rm = np.zeros((NR, HEADS * NP), np.float32)
    Em = np.zeros((HEADS * NP, NR), np.float32)
    for j in range(NP):
        for h in range(HEADS):
            for u in range(HD):
                Rm[j * RPP + h * HD + u, h * NP + j] = 1.0
                Em[h * NP + j, j * RPP + h * HD + u] = 1.0
    Gm = np.kron(np.eye(HEADS, dtype=np.float32), np.ones((NP, NP), np.float32))
    Fm = np.tile(np.eye(RPP, dtype=np.float32), (NP, 1))
    Mm = np.kron(np.eye(NP, dtype=np.float32),
                 np.ones((RPP, RPP), np.float32) / RPP)
    Bm = np.kron(np.eye(NP, dtype=np.float32), np.ones((RPP, RPP), np.float32))
    Pm = np.zeros((NR, NR), np.float32)
    for p in range(NP):
        for r in range(RPP):
            Pm[p * RPP + r, r * NP + p] = 1.0
    return Rm, Em, Gm, Fm, Mm, Bm, Pm


_PE = _pe_lane()
_R, _E, _G, _F, _MAVG, _BMASK, _PERM = _selectors()
_X8 = np.kron(np.eye(NP, dtype=np.float32), np.ones((1, RPP), np.float32))
_T16 = np.tile(np.eye(RPP, dtype=np.float32), (1, NP))


def _k1_body(d_ref, w1_ref, b1_ref, w2_ref, b2_ref,
             lo_ref, hi_ref, x8_ref, rf_ref, out_ref):
    f32 = jnp.float32
    dgT = lambda a, b: jax.lax.dot_general(a, b, (((0,), (0,)), ((), ())),
                                           preferred_element_type=f32)
    for bb in range(K1_BB):
        datT = d_ref[bb]
        valsT = datT[:DIM, :]
        maskT = datT[DIM:2 * DIM, :]
        trow = datT[2 * DIM:2 * DIM + 1, :]
        vT = valsT * maskT
        h1T = jnp.maximum(dgT(w1_ref[...], vT) + b1_ref[...], 0.0)
        hT = dgT(w2_ref[...], h1T) + b2_ref[...]
        obs = jnp.sum(maskT, axis=0, keepdims=True) > 0.0
        tm8 = ((trow >= lo_ref[...]) & (trow <= hi_ref[...]) & obs)
        tm8t = jnp.transpose(tm8.astype(f32), (1, 0))
        tm = jnp.dot(tm8t, x8_ref[...], preferred_element_type=f32)
        tcol = jnp.transpose(trow, (1, 0))
        u = rf_ref[...] - tcol
        w = jnp.exp(-100.0 * (u * u)) * tm
        haug = jnp.concatenate([hT, jnp.ones((1, T), f32)], axis=0)
        repz = jnp.dot(haug, w, preferred_element_type=f32)
        z = repz[DIM:DIM + 1, :]
        rz = 1.0 / jnp.where(z > 0.0, z, 1.0)
        out_ref[bb] = repz[:DIM, :] * rz


def _k2_body(x_ref, wq_ref, wk_ref, wv_ref, wo_ref, wf1_ref, bf1_ref,
             wf2_ref, bf2_ref, g1_ref, c1_ref, g2_ref, c2_ref, gf_ref, cf_ref,
             wl_ref, bl_ref, pe_ref, r_ref, e_ref, gg_ref, f_ref, mavg_ref,
             bm_ref, pm_ref, t16_ref, out_ref):
    f32 = jnp.float32
    dot = lambda a, b: jnp.dot(a, b, preferred_element_type=f32)
    lane8 = lambda a: jnp.concatenate([a] * NP, axis=1)
    def bdiag(w):
        wt = dot(w, t16_ref[...])
        return jnp.concatenate([wt] * NP, axis=0) * bm_ref[...]
    x = x_ref[...] + pe_ref[...]
    inv_sqrt_hd = 1.0 / math.sqrt(float(HD))

    g1t = [lane8(g1_ref[l:l + 1, :]) for l in range(LAYERS)]
    c1t = [lane8(c1_ref[l:l + 1, :]) for l in range(LAYERS)]
    g2t = [lane8(g2_ref[l:l + 1, :]) for l in range(LAYERS)]
    c2t = [lane8(c2_ref[l:l + 1, :]) for l in range(LAYERS)]
    fwo = [dot(f_ref[...], wo_ref[l]) for l in range(LAYERS)]

    def ln(z, g, c):
        m = dot(z, mavg_ref[...])
        ex2 = dot(z * z, mavg_ref[...])
        va = ex2 - m * m
        return (z - m) * jax.lax.rsqrt(va + 1e-5) * g + c

    for l in range(LAYERS):
        q = dot(x, bdiag(wq_ref[l]))
        k = dot(x, bdiag(wk_ref[l]))
        v = dot(x, bdiag(wv_ref[l]))
        o_parts = []
        for i in range(NP):
            qi = lane8(q[:, i * RPP:(i + 1) * RPP])
            si = dot(qi * k, r_ref[...])
            ei = jnp.exp(jnp.minimum(si, 60.0))
            di = dot(ei, gg_ref[...])
            pi = ei / di
            pex = dot(pi, e_ref[...])
            o_parts.append(dot(pex * v, fwo[l]))
        x = x + jnp.concatenate(o_parts, axis=1)
        x = ln(x, g1t[l], c1t[l])
        y_parts = []
        for p in range(NP):
            xp = x[:, p * RPP:(p + 1) * RPP]
            yp = jax.nn.gelu(dot(xp, wf1_ref[l]) + bf1_ref[l:l + 1, :])
            y_parts.append(dot(yp, wf2_ref[l]) + bf2_ref[l:l + 1, :])
        y = jnp.concatenate(y_parts, axis=1)
        x = ln(x + y, g2t[l], c2t[l])
    x = ln(x, lane8(gf_ref[...]), lane8(cf_ref[...]))
    wperm = dot(pm_ref[...], wl_ref[...])
    out_ref[...] = dot(x, wperm) + bl_ref[...]


@functools.partial(jax.jit, static_argnums=())
def kernel(data, W1, b1, W2, b2, Wq, Wk, Wv, Wo, Wf1, bf1, Wf2, bf2,
           ln1g, ln1b, ln2g, ln2b, lnfg, lnfb, Wlin, blin):
    f32 = jnp.float32
    edges = np.linspace(0.0, OBS, NP + 1)
    lo = jnp.asarray(edges[:NP], f32).reshape(NP, 1)
    hi = jnp.asarray(edges[1:], f32).reshape(NP, 1)
    refs = jnp.linspace(0.0, OBS, NR, dtype=f32).reshape(1, NR)
    dataT = jnp.transpose(data, (0, 2, 1))

    rep = pl.pallas_call(
        _k1_body,
        grid=(B // K1_BB,),
        in_specs=[
            pl.BlockSpec((K1_BB, 2 * DIM + 1, T), lambda b: (b, 0, 0)),
            pl.BlockSpec((DIM, LAT), lambda b: (0, 0)),
            pl.BlockSpec((LAT, 1), lambda b: (0, 0)),
            pl.BlockSpec((LAT, DIM), lambda b: (0, 0)),
            pl.BlockSpec((DIM, 1), lambda b: (0, 0)),
            pl.BlockSpec((NP, 1), lambda b: (0, 0)),
            pl.BlockSpec((NP, 1), lambda b: (0, 0)),
            pl.BlockSpec((NP, NR), lambda b: (0, 0)),
            pl.BlockSpec((1, NR), lambda b: (0, 0)),
        ],
        out_specs=pl.BlockSpec((K1_BB, DIM, NR), lambda b: (b, 0, 0)),
        out_shape=jax.ShapeDtypeStruct((B, DIM, NR), f32),
    )(dataT, W1, b1.reshape(LAT, 1), W2, b2.reshape(DIM, 1), lo, hi,
      jnp.asarray(_X8), refs)

    xflat = rep.reshape(ROWS, NR)

    full = lambda *sh: pl.BlockSpec(sh, lambda: tuple(0 for _ in sh))
    out = pl.pallas_call(
        _k2_body,
        in_specs=[
            full(ROWS, NR),
            full(LAYERS, RPP, RPP), full(LAYERS, RPP, RPP),
            full(LAYERS, RPP, RPP), full(LAYERS, RPP, RPP),
            full(LAYERS, RPP, DFF), full(LAYERS, DFF),
            full(LAYERS, DFF, RPP), full(LAYERS, RPP),
            full(LAYERS, RPP), full(LAYERS, RPP),
            full(LAYERS, RPP), full(LAYERS, RPP),
            full(1, RPP), full(1, RPP),
            full(NR, PRED), full(1, PRED),
            full(1, NR),
            full(NR, HEADS * NP), full(HEADS * NP, NR),
            full(HEADS * NP, HEADS * NP), full(NR, RPP),
            full(NR, NR), full(NR, NR), full(NR, NR),
            full(RPP, NR),
        ],
        out_specs=full(ROWS, PRED),
        out_shape=jax.ShapeDtypeStruct((ROWS, PRED), f32),
    )(xflat, Wq, Wk, Wv, Wo, Wf1, bf1, Wf2, bf2,
      ln1g, ln1b, ln2g, ln2b, lnfg.reshape(1, RPP), lnfb.reshape(1, RPP),
      Wlin, blin.reshape(1, PRED),
      jnp.asarray(_PE), jnp.asarray(_R / math.sqrt(float(HD))),
      jnp.asarray(_E), jnp.asarray(_G),
      jnp.asarray(_F), jnp.asarray(_MAVG), jnp.asarray(_BMASK),
      jnp.asarray(_PERM), jnp.asarray(_T16))

    return jnp.transpose(out.reshape(B, DIM, PRED), (0, 2, 1))
